# baseline (speedup 1.0000x reference)
import functools

import jax
import jax.numpy as jnp
from jax import lax
from jax.experimental import pallas as pl
from jax.experimental.pallas import tpu as pltpu
from jax.experimental.pallas import tpu_sc as plsc

_V = 65
_B, _L = 4096, 200
_T = _B * _L
_NC, _NS, _LN = 2, 16, 16
_NW = _NC * _NS
_TPW = _T // _NW
_CH = 128
_NCH = _TPW // _CH


def _head_kernel(table_ref, w_ref, b_ref, m_ref, lse_ref):
    m = jnp.dot(table_ref[...], w_ref[...],
                preferred_element_type=jnp.float32) + b_ref[...]
    m_ref[...] = m
    lanes = jax.lax.broadcasted_iota(jnp.int32, (_V, 128), 1)
    mm = jnp.where(lanes < _V, m, -jnp.inf)
    mx = jnp.max(mm, axis=1, keepdims=True)
    lse_ref[...] = mx + jnp.log(
        jnp.sum(jnp.where(lanes < _V, jnp.exp(mm - mx), 0.0),
                axis=1, keepdims=True))


_mesh = plsc.VectorSubcoreMesh(core_axis_name="c", subcore_axis_name="s",
                               num_cores=_NC, num_subcores=_NS)


@functools.partial(
    pl.kernel,
    compiler_params=pltpu.CompilerParams(needs_layout_passes=False),
    out_type=(
        jax.ShapeDtypeStruct((_T, _V), jnp.float32),
        jax.ShapeDtypeStruct((_NW, _LN), jnp.float32),
    ),
    mesh=_mesh,
    scratch_types=[
        pltpu.VMEM((_TPW,), jnp.int32),
        pltpu.VMEM((_TPW,), jnp.int32),
        pltpu.VMEM((80,), jnp.float32),
        pltpu.VMEM((_CH, 128), jnp.float32),
        pltpu.VMEM((_CH, 128), jnp.float32),
        pltpu.VMEM((_CH, _V), jnp.float32),
        pltpu.VMEM((_CH, _V), jnp.float32),
        pltpu.VMEM((_LN,), jnp.float32),
        pltpu.SemaphoreType.DMA,
        pltpu.SemaphoreType.DMA,
    ],
)
def _sc_gather(m_hbm, lse_hbm, x_hbm, t_hbm, out_hbm, parts_hbm,
               xf_v, tf_v, lse_v, rows0, rows1, cmp0, cmp1, acc_v, g0, g1):
    wid = lax.axis_index("s") * _NC + lax.axis_index("c")
    base = wid * _TPW
    pltpu.sync_copy(x_hbm.at[pl.ds(base, _TPW)], xf_v)
    pltpu.sync_copy(t_hbm.at[pl.ds(base, _TPW)], tf_v)
    pltpu.sync_copy(lse_hbm, lse_v)

    jidx = lax.broadcasted_iota(jnp.int32, (_LN,), 0)

    def chunk_loss(rows, c, acc):
        def kbody(k, acc):
            off = c * _CH + k * _LN
            xv = xf_v[pl.ds(off, _LN)]
            tv = tf_v[pl.ds(off, _LN)]
            lsev = plsc.load_gather(lse_v, [xv])
            tlog = plsc.load_gather(rows, [jidx + k * _LN, tv])
            return acc + (lsev - tlog)
        return lax.fori_loop(0, _CH // _LN, kbody, acc)

    def compact(rows, cmp):
        def gbody(g, carry):
            jb = g * _LN
            for j in range(_LN):
                for o in (0, 16, 32, 48, 49):
                    cmp[jb + j, pl.ds(o, _LN)] = rows[jb + j, pl.ds(o, _LN)]
            return carry
        lax.fori_loop(0, _CH // _LN, gbody, 0)

    def body(i, acc):
        cps = []
        for bi, (rows, gs) in enumerate(((rows0, g0), (rows1, g1))):
            c = 2 * i + bi
            cps.append(
                pltpu.async_copy(m_hbm.at[xf_v.at[pl.ds(c * _CH, _CH)]],
                                 rows, gs))
        for bi, (rows, cmp) in enumerate(((rows0, cmp0), (rows1, cmp1))):
            c = 2 * i + bi
            cps[bi].wait()
            acc = chunk_loss(rows, c, acc)
            compact(rows, cmp)
            pltpu.sync_copy(cmp, out_hbm.at[pl.ds(base + c * _CH, _CH)])
        return acc

    acc = lax.fori_loop(0, _NCH // 2, body, jnp.zeros((_LN,), jnp.float32))
    acc_v[...] = acc
    pltpu.sync_copy(acc_v, parts_hbm.at[wid])


def kernel(x, targets, table, W, b):
    w128 = jnp.pad(W, ((0, 0), (0, 128 - _V)))
    b128 = jnp.pad(b, (0, 128 - _V)).reshape(1, 128)
    m, lse = pl.pallas_call(
        _head_kernel,
        out_shape=(
            jax.ShapeDtypeStruct((_V, 128), jnp.float32),
            jax.ShapeDtypeStruct((_V, 1), jnp.float32),
        ),
    )(table, w128, b128)

    lse80 = jnp.pad(lse[:, 0], (0, 80 - _V))
    logits_flat, parts = _sc_gather(m, lse80, x.reshape(_T), targets.reshape(_T))
    loss = jnp.sum(parts) / _T
    return (logits_flat.reshape(_B, _L, _V), loss)
